# Initial kernel scaffold; baseline (speedup 1.0000x reference)
import jax
import jax.numpy as jnp
from jax.experimental import pallas as pl


def kernel(x):
    raise NotImplementedError("write your pallas kernel here")


# speedup vs baseline: 45.7840x; 45.7840x over previous
import jax
import jax.numpy as jnp
import numpy as np
from jax.experimental import pallas as pl

_NUM_PATCHES = 96
_PATCH_SIZE = 256

_L_BLOCK = 2048


def _check_full_coverage(L: int) -> None:
    start = np.round(np.linspace(0.0, L - _PATCH_SIZE, _NUM_PATCHES)).astype(np.int64)
    idx = (start[:, None] + np.arange(_PATCH_SIZE)[None, :]).reshape(-1)
    count = np.zeros((L,), dtype=np.int64)
    np.add.at(count, idx, 1)
    assert count.min() >= 1, "patch layout leaves uncovered positions"


def _transpose_block_kernel(x_ref, o_ref):
    o_ref[0] = x_ref[0].T


def kernel(x):
    B, C, L = x.shape
    _check_full_coverage(L)
    lb = _L_BLOCK if L % _L_BLOCK == 0 else L
    return pl.pallas_call(
        _transpose_block_kernel,
        grid=(B, L // lb),
        in_specs=[pl.BlockSpec((1, C, lb), lambda b, l: (b, 0, l))],
        out_specs=pl.BlockSpec((1, lb, C), lambda b, l: (b, l, 0)),
        out_shape=jax.ShapeDtypeStruct((B, L, C), x.dtype),
    )(x)


# speedup vs baseline: 64.4887x; 1.4085x over previous
import jax
import jax.numpy as jnp
import numpy as np
from jax.experimental import pallas as pl

_NUM_PATCHES = 96
_PATCH_SIZE = 256

_L_BLOCK = 4096


def _check_full_coverage(L: int) -> None:
    start = np.round(np.linspace(0.0, L - _PATCH_SIZE, _NUM_PATCHES)).astype(np.int64)
    idx = (start[:, None] + np.arange(_PATCH_SIZE)[None, :]).reshape(-1)
    count = np.zeros((L,), dtype=np.int64)
    np.add.at(count, idx, 1)
    assert count.min() >= 1, "patch layout leaves uncovered positions"


def _transpose_block_kernel(x_ref, o_ref):
    o_ref[0] = x_ref[0].T


def kernel(x):
    B, C, L = x.shape
    _check_full_coverage(L)
    lb = _L_BLOCK if L % _L_BLOCK == 0 else L
    return pl.pallas_call(
        _transpose_block_kernel,
        grid=(B, L // lb),
        in_specs=[pl.BlockSpec((1, C, lb), lambda b, l: (b, 0, l))],
        out_specs=pl.BlockSpec((1, lb, C), lambda b, l: (b, l, 0)),
        out_shape=jax.ShapeDtypeStruct((B, L, C), x.dtype),
    )(x)


# speedup vs baseline: 74.0866x; 1.1488x over previous
import jax
import jax.numpy as jnp
import numpy as np
from jax.experimental import pallas as pl

_NUM_PATCHES = 96
_PATCH_SIZE = 256

_L_BLOCK = 8192


def _check_full_coverage(L: int) -> None:
    start = np.round(np.linspace(0.0, L - _PATCH_SIZE, _NUM_PATCHES)).astype(np.int64)
    idx = (start[:, None] + np.arange(_PATCH_SIZE)[None, :]).reshape(-1)
    count = np.zeros((L,), dtype=np.int64)
    np.add.at(count, idx, 1)
    assert count.min() >= 1, "patch layout leaves uncovered positions"


def _transpose_block_kernel(x_ref, o_ref):
    o_ref[0] = x_ref[0].T


def kernel(x):
    B, C, L = x.shape
    _check_full_coverage(L)
    lb = _L_BLOCK if L % _L_BLOCK == 0 else L
    return pl.pallas_call(
        _transpose_block_kernel,
        grid=(B, L // lb),
        in_specs=[pl.BlockSpec((1, C, lb), lambda b, l: (b, 0, l))],
        out_specs=pl.BlockSpec((1, lb, C), lambda b, l: (b, l, 0)),
        out_shape=jax.ShapeDtypeStruct((B, L, C), x.dtype),
    )(x)


# speedup vs baseline: 75.9123x; 1.0246x over previous
import jax
import jax.numpy as jnp
import numpy as np
from jax.experimental import pallas as pl

_NUM_PATCHES = 96
_PATCH_SIZE = 256

_L_BLOCK = 8192


def _check_full_coverage(L: int) -> None:
    start = np.round(np.linspace(0.0, L - _PATCH_SIZE, _NUM_PATCHES)).astype(np.int64)
    idx = (start[:, None] + np.arange(_PATCH_SIZE)[None, :]).reshape(-1)
    count = np.zeros((L,), dtype=np.int64)
    np.add.at(count, idx, 1)
    assert count.min() >= 1, "patch layout leaves uncovered positions"


def _transpose_block_kernel(x_ref, o_ref):
    for i in range(x_ref.shape[0]):
        o_ref[i] = x_ref[i].T


def kernel(x):
    B, C, L = x.shape
    _check_full_coverage(L)
    lb = _L_BLOCK if L % _L_BLOCK == 0 else L
    bb = 2 if B % 2 == 0 else 1
    return pl.pallas_call(
        _transpose_block_kernel,
        grid=(B // bb, L // lb),
        in_specs=[pl.BlockSpec((bb, C, lb), lambda b, l: (b, 0, l))],
        out_specs=pl.BlockSpec((bb, lb, C), lambda b, l: (b, l, 0)),
        out_shape=jax.ShapeDtypeStruct((B, L, C), x.dtype),
    )(x)


# speedup vs baseline: 76.0388x; 1.0017x over previous
import jax
import jax.numpy as jnp
import numpy as np
from jax.experimental import pallas as pl

_NUM_PATCHES = 96
_PATCH_SIZE = 256

_L_BLOCK = 4096


def _check_full_coverage(L: int) -> None:
    start = np.round(np.linspace(0.0, L - _PATCH_SIZE, _NUM_PATCHES)).astype(np.int64)
    idx = (start[:, None] + np.arange(_PATCH_SIZE)[None, :]).reshape(-1)
    count = np.zeros((L,), dtype=np.int64)
    np.add.at(count, idx, 1)
    assert count.min() >= 1, "patch layout leaves uncovered positions"


def _transpose_block_kernel(x_ref, o_ref):
    for i in range(x_ref.shape[0]):
        o_ref[i] = x_ref[i].T


def kernel(x):
    B, C, L = x.shape
    _check_full_coverage(L)
    lb = _L_BLOCK if L % _L_BLOCK == 0 else L
    bb = 4 if B % 4 == 0 else 1
    return pl.pallas_call(
        _transpose_block_kernel,
        grid=(B // bb, L // lb),
        in_specs=[pl.BlockSpec((bb, C, lb), lambda b, l: (b, 0, l))],
        out_specs=pl.BlockSpec((bb, lb, C), lambda b, l: (b, l, 0)),
        out_shape=jax.ShapeDtypeStruct((B, L, C), x.dtype),
    )(x)
